# Initial kernel scaffold; baseline (speedup 1.0000x reference)
import jax
import jax.numpy as jnp
from jax.experimental import pallas as pl


def kernel(x, pos, edge_index, batch, W1, b1, W2, b2, Wp, bp, Wz, bz):
    raise NotImplementedError("write your pallas kernel here")


# speedup vs baseline: 18.7003x; 18.7003x over previous
import functools

import jax
import jax.numpy as jnp
from jax import lax
from jax.experimental import pallas as pl
from jax.experimental.pallas import tpu as pltpu
from jax.experimental.pallas import tpu_sc as plsc

N = 100000
E = 1600000
G = 256
H = 16

NC = 2
NSUBC = 16
NW = NC * NSUBC

PER_W = 51200
E_PAD = NW * PER_W
CHUNK = 2048
NCH = PER_W // CHUNK
SUB = 128
NSTREAM = CHUNK // SUB

NS = 100352
RPT = NS // NSUBC

BR = 512
BN = 2000

_SC_PARAMS = pltpu.CompilerParams(use_tc_tiling_on_sc=False)


def _sc_mesh():
    return plsc.VectorSubcoreMesh(core_axis_name="c", subcore_axis_name="s",
                                  num_cores=NC, num_subcores=NSUBC)


def _sc_gather_body(src_hbm, dst_hbm, xp_hbm, es_hbm, ed_hbm,
                    sidx, didx, bs, bd, sem):
    cid = lax.axis_index("c")
    sid = lax.axis_index("s")
    wid = sid * NC + cid

    def chunk(ci, carry):
        rbase = wid * (PER_W // SUB) + ci * NSTREAM
        ebase = wid * PER_W + ci * CHUNK
        pltpu.sync_copy(src_hbm.at[pl.ds(rbase, NSTREAM)], sidx)
        pltpu.sync_copy(dst_hbm.at[pl.ds(rbase, NSTREAM)], didx)
        handles = []
        for j in range(NSTREAM):
            handles.append(pltpu.async_copy(
                xp_hbm.at[sidx.at[j]], bs.at[pl.ds(j * SUB, SUB)], sem))
            handles.append(pltpu.async_copy(
                xp_hbm.at[didx.at[j]], bd.at[pl.ds(j * SUB, SUB)], sem))
        for h in handles:
            h.wait()
        pltpu.sync_copy(bs, es_hbm.at[pl.ds(ebase, CHUNK)])
        pltpu.sync_copy(bd, ed_hbm.at[pl.ds(ebase, CHUNK)])
        return carry

    lax.fori_loop(0, NCH, chunk, 0)


@functools.cache
def _sc_gather():
    return pl.kernel(
        _sc_gather_body,
        out_type=[
            jax.ShapeDtypeStruct((E_PAD, 8), jnp.float32),
            jax.ShapeDtypeStruct((E_PAD, 8), jnp.float32),
        ],
        mesh=_sc_mesh(),
        compiler_params=_SC_PARAMS,
        scratch_types=[
            pltpu.VMEM((NSTREAM, SUB), jnp.int32),
            pltpu.VMEM((NSTREAM, SUB), jnp.int32),
            pltpu.VMEM((CHUNK, 8), jnp.float32),
            pltpu.VMEM((CHUNK, 8), jnp.float32),
            pltpu.SemaphoreType.DMA,
        ],
    )


def _mlp_body(es_ref, ed_ref, w1i_ref, w1j_ref, wc_ref, ssel_ref, b1_ref,
              w2_ref, b2_ref, out_ref):
    eg = es_ref[...]
    dg = ed_ref[...]
    rel = eg - dg
    d2 = jnp.dot(rel * rel, ssel_ref[...], preferred_element_type=jnp.float32)
    dist = jnp.sqrt(d2)
    pre = (jnp.dot(dg, w1i_ref[...], preferred_element_type=jnp.float32)
           + jnp.dot(eg, w1j_ref[...], preferred_element_type=jnp.float32)
           + jnp.dot(dist, wc_ref[...], preferred_element_type=jnp.float32)
           + b1_ref[...])
    h1 = pre * jax.nn.sigmoid(pre)
    out_ref[...] = (jnp.dot(h1, w2_ref[...],
                            preferred_element_type=jnp.float32)
                    + b2_ref[...])


def _edge_mlp(esg, edg, w1i, w1j, wc, ssel, b1t, w2bd, b2t):
    grid = (E_PAD // 16 // BR,)
    return pl.pallas_call(
        _mlp_body,
        grid=grid,
        in_specs=[
            pl.BlockSpec((BR, 128), lambda i: (i, 0)),
            pl.BlockSpec((BR, 128), lambda i: (i, 0)),
            pl.BlockSpec((128, 256), lambda i: (0, 0)),
            pl.BlockSpec((128, 256), lambda i: (0, 0)),
            pl.BlockSpec((16, 256), lambda i: (0, 0)),
            pl.BlockSpec((128, 16), lambda i: (0, 0)),
            pl.BlockSpec((1, 256), lambda i: (0, 0)),
            pl.BlockSpec((256, 256), lambda i: (0, 0)),
            pl.BlockSpec((1, 256), lambda i: (0, 0)),
        ],
        out_specs=pl.BlockSpec((BR, 256), lambda i: (i, 0)),
        out_shape=jax.ShapeDtypeStruct((E_PAD // 16, 256), jnp.float32),
    )(esg, edg, w1i, w1j, wc, ssel, b1t, w2bd, b2t)


SCHUNK = 1024
SNCH = PER_W // SCHUNK
SNSTREAM = SCHUNK // SUB


def _sc_scatter_body(msg_hbm, dstT_hbm, zs_hbm, sums_hbm,
                     msgv, didx, acc, sem):
    cid = lax.axis_index("c")
    sid = lax.axis_index("s")
    wid = sid * NC + cid

    pltpu.sync_copy(zs_hbm, acc.at[pl.ds(sid * RPT, RPT)])
    plsc.subcore_barrier()

    def chunk(ci, carry):
        rbase = wid * (PER_W // SUB) + ci * SNSTREAM
        ebase = wid * PER_W + ci * SCHUNK
        pltpu.sync_copy(msg_hbm.at[pl.ds(ebase, SCHUNK)], msgv)
        pltpu.sync_copy(dstT_hbm.at[pl.ds(rbase, SNSTREAM)], didx)
        for j in range(SNSTREAM):
            pltpu.sync_copy(msgv.at[pl.ds(j * SUB, SUB)],
                            acc.at[didx.at[j]], add=True)
        return carry

    lax.fori_loop(0, SNCH, chunk, 0)
    plsc.subcore_barrier()
    pltpu.sync_copy(acc.at[pl.ds(sid * RPT, RPT)],
                    sums_hbm.at[cid, pl.ds(sid * RPT, RPT)])


@functools.cache
def _sc_scatter():
    return pl.kernel(
        _sc_scatter_body,
        out_type=[jax.ShapeDtypeStruct((NC, NS, H), jnp.float32)],
        mesh=_sc_mesh(),
        compiler_params=_SC_PARAMS,
        scratch_types=[
            pltpu.VMEM((SCHUNK, H), jnp.float32),
            pltpu.VMEM((SNSTREAM, SUB), jnp.int32),
            pltpu.VMEM_SHARED((NS, H), jnp.float32),
            pltpu.SemaphoreType.DMA,
        ],
    )


def _sc_count_body(dstT_hbm, ones_hbm, zc_hbm, cnts_hbm,
                   didx, onesv, accc, sem):
    cid = lax.axis_index("c")
    sid = lax.axis_index("s")
    wid = sid * NC + cid

    pltpu.sync_copy(zc_hbm, accc.at[pl.ds(sid * RPT, RPT)])
    pltpu.sync_copy(ones_hbm, onesv)
    plsc.subcore_barrier()

    def outer(ci, carry):
        rbase = wid * (PER_W // SUB) + ci * NSTREAM
        pltpu.sync_copy(dstT_hbm.at[pl.ds(rbase, NSTREAM)], didx)
        for j in range(NSTREAM):
            pltpu.sync_copy(onesv, accc.at[didx.at[j]], add=True)
        return carry

    lax.fori_loop(0, NCH, outer, 0)
    plsc.subcore_barrier()
    pltpu.sync_copy(accc.at[pl.ds(sid * RPT, RPT)],
                    cnts_hbm.at[cid, pl.ds(sid * RPT, RPT)])


@functools.cache
def _sc_count():
    return pl.kernel(
        _sc_count_body,
        out_type=[jax.ShapeDtypeStruct((NC, NS, H), jnp.float32)],
        mesh=_sc_mesh(),
        compiler_params=_SC_PARAMS,
        scratch_types=[
            pltpu.VMEM((NSTREAM, SUB), jnp.int32),
            pltpu.VMEM((SUB, H), jnp.float32),
            pltpu.VMEM_SHARED((NS, H), jnp.float32),
            pltpu.SemaphoreType.DMA,
        ],
    )


def _final_body(s0_ref, s1_ref, c0_ref, c1_ref, batch_ref, wp_ref, bp_ref,
                wz_ref, bz8_ref, s_ref, z_ref):
    tot = s0_ref[...] + s1_ref[...]
    cnt = jnp.maximum(c0_ref[...] + c1_ref[...], 1.0)
    h = jnp.maximum(tot / cnt, 0.0)
    logits = jnp.dot(h, wp_ref[...], preferred_element_type=jnp.float32) \
        + bp_ref[...]
    mx = jnp.max(logits, axis=1, keepdims=True)
    ex = jnp.exp(logits - mx)
    s = ex / jnp.sum(ex, axis=1, keepdims=True)
    s_ref[...] = s
    hz = jnp.dot(h, wz_ref[...], preferred_element_type=jnp.float32)
    wz8 = jnp.concatenate([s[:, 0:1] * hz, s[:, 1:2] * hz], axis=1)
    oh = (batch_ref[...] == lax.broadcasted_iota(jnp.int32, (BN, G), 1)
          ).astype(jnp.float32)
    part = lax.dot_general(oh, wz8, (((0,), (0,)), ((), ())),
                           preferred_element_type=jnp.float32)

    @pl.when(pl.program_id(0) == 0)
    def _():
        z_ref[...] = jnp.broadcast_to(bz8_ref[...], (G, 8))

    z_ref[...] += part


def _finalize(s0, s1, c0, c1, batchc, wp, bpr, wz, bz8):
    grid = (N // BN,)
    return pl.pallas_call(
        _final_body,
        grid=grid,
        in_specs=[
            pl.BlockSpec((BN, H), lambda i: (i, 0)),
            pl.BlockSpec((BN, H), lambda i: (i, 0)),
            pl.BlockSpec((BN, 1), lambda i: (i, 0)),
            pl.BlockSpec((BN, 1), lambda i: (i, 0)),
            pl.BlockSpec((BN, 1), lambda i: (i, 0)),
            pl.BlockSpec((H, 2), lambda i: (0, 0)),
            pl.BlockSpec((1, 2), lambda i: (0, 0)),
            pl.BlockSpec((H, 4), lambda i: (0, 0)),
            pl.BlockSpec((1, 8), lambda i: (0, 0)),
        ],
        out_specs=[
            pl.BlockSpec((BN, 2), lambda i: (i, 0)),
            pl.BlockSpec((G, 8), lambda i: (0, 0)),
        ],
        out_shape=[
            jax.ShapeDtypeStruct((N, 2), jnp.float32),
            jax.ShapeDtypeStruct((G, 8), jnp.float32),
        ],
    )(s0, s1, c0, c1, batchc, wp, bpr, wz, bz8)


def kernel(x, pos, edge_index, batch, W1, b1, W2, b2, Wp, bp, Wz, bz):
    pad = E_PAD - E
    src = edge_index[0]
    dst = edge_index[1]
    zpad = jnp.zeros((pad,), jnp.int32)
    srcp = jnp.concatenate([src, zpad]).reshape(E_PAD // SUB, SUB)
    dstp = jnp.concatenate([dst, zpad]).reshape(E_PAD // SUB, SUB)
    dstT = jnp.concatenate([dst, jnp.full((pad,), N, jnp.int32)]
                           ).reshape(E_PAD // SUB, SUB)
    xp = jnp.concatenate([x, pos, jnp.zeros((N, 1), jnp.float32)], axis=1)

    es, ed = _sc_gather()(srcp, dstp, xp)
    esg = es.reshape(E_PAD // 16, 128)
    edg = ed.reshape(E_PAD // 16, 128)

    eye = jnp.eye(16, dtype=jnp.float32)
    zeros4 = jnp.zeros((4, H), jnp.float32)
    w1i = jnp.kron(eye, jnp.concatenate([W1[0:4], zeros4], axis=0))
    w1j = jnp.kron(eye, jnp.concatenate([W1[4:8], zeros4], axis=0))
    wc = jnp.kron(eye, W1[8:9])
    ssel = jnp.kron(eye, jnp.array([[0.], [0.], [0.], [0.],
                                    [1.], [1.], [1.], [0.]], jnp.float32))
    b1t = jnp.tile(b1, 16).reshape(1, 256)
    w2bd = jnp.kron(eye, W2)
    b2t = jnp.tile(b2, 16).reshape(1, 256)

    msgg = _edge_mlp(esg, edg, w1i, w1j, wc, ssel, b1t, w2bd, b2t)
    msg = msgg.reshape(E_PAD, H)

    ones = jnp.ones((SUB, H), jnp.float32)
    zs = jnp.zeros((RPT, H), jnp.float32)
    zc = jnp.zeros((RPT, H), jnp.float32)
    (sums,) = _sc_scatter()(msg, dstT, zs)
    (cnts,) = _sc_count()(dstT, ones, zc)

    s, z8 = _finalize(sums[0, :N], sums[1, :N],
                      cnts[0, :N, 0:1], cnts[1, :N, 0:1],
                      batch.reshape(N, 1),
                      Wp, bp.reshape(1, 2), Wz,
                      jnp.concatenate([bz, bz]).reshape(1, 8))
    z = z8.reshape(G, 2, 4)
    return (z, s)


# speedup vs baseline: 18.7415x; 1.0022x over previous
import functools

import jax
import jax.numpy as jnp
from jax import lax
from jax.experimental import pallas as pl
from jax.experimental.pallas import tpu as pltpu
from jax.experimental.pallas import tpu_sc as plsc

N = 100000
E = 1600000
G = 256
H = 16

NC = 2
NSUBC = 16
NW = NC * NSUBC

PER_W = 51200
E_PAD = NW * PER_W
CHUNK = 2048
NCH = PER_W // CHUNK
SUB = 128
NSTREAM = CHUNK // SUB

NS = 100352
RPT = NS // NSUBC

BR = 512
BN = 2000

_SC_PARAMS = pltpu.CompilerParams(use_tc_tiling_on_sc=False)


def _sc_mesh():
    return plsc.VectorSubcoreMesh(core_axis_name="c", subcore_axis_name="s",
                                  num_cores=NC, num_subcores=NSUBC)


def _sc_gather_body(src_hbm, dst_hbm, xp_hbm, es_hbm, ed_hbm,
                    sidx, didx, bs, bd, sem):
    cid = lax.axis_index("c")
    sid = lax.axis_index("s")
    wid = sid * NC + cid

    def chunk(ci, carry):
        ebase = wid * PER_W + ci * CHUNK
        pltpu.sync_copy(src_hbm.at[pl.ds(ebase, CHUNK)], sidx)
        pltpu.sync_copy(dst_hbm.at[pl.ds(ebase, CHUNK)], didx)
        h1 = pltpu.async_copy(xp_hbm.at[sidx], bs, sem)
        h2 = pltpu.async_copy(xp_hbm.at[didx], bd, sem)
        h1.wait()
        h2.wait()
        pltpu.sync_copy(bs, es_hbm.at[pl.ds(ebase, CHUNK)])
        pltpu.sync_copy(bd, ed_hbm.at[pl.ds(ebase, CHUNK)])
        return carry

    lax.fori_loop(0, NCH, chunk, 0)


@functools.cache
def _sc_gather():
    return pl.kernel(
        _sc_gather_body,
        out_type=[
            jax.ShapeDtypeStruct((E_PAD, 8), jnp.float32),
            jax.ShapeDtypeStruct((E_PAD, 8), jnp.float32),
        ],
        mesh=_sc_mesh(),
        compiler_params=_SC_PARAMS,
        scratch_types=[
            pltpu.VMEM((CHUNK,), jnp.int32),
            pltpu.VMEM((CHUNK,), jnp.int32),
            pltpu.VMEM((CHUNK, 8), jnp.float32),
            pltpu.VMEM((CHUNK, 8), jnp.float32),
            pltpu.SemaphoreType.DMA,
        ],
    )


def _mlp_body(es_ref, ed_ref, w1i_ref, w1j_ref, wc_ref, ssel_ref, b1_ref,
              w2_ref, b2_ref, out_ref):
    eg = es_ref[...]
    dg = ed_ref[...]
    rel = eg - dg
    d2 = jnp.dot(rel * rel, ssel_ref[...], preferred_element_type=jnp.float32)
    dist = jnp.sqrt(d2)
    pre = (jnp.dot(dg, w1i_ref[...], preferred_element_type=jnp.float32)
           + jnp.dot(eg, w1j_ref[...], preferred_element_type=jnp.float32)
           + jnp.dot(dist, wc_ref[...], preferred_element_type=jnp.float32)
           + b1_ref[...])
    h1 = pre * jax.nn.sigmoid(pre)
    out_ref[...] = (jnp.dot(h1, w2_ref[...],
                            preferred_element_type=jnp.float32)
                    + b2_ref[...])


def _edge_mlp(esg, edg, w1i, w1j, wc, ssel, b1t, w2bd, b2t):
    grid = (E_PAD // 16 // BR,)
    return pl.pallas_call(
        _mlp_body,
        grid=grid,
        in_specs=[
            pl.BlockSpec((BR, 128), lambda i: (i, 0)),
            pl.BlockSpec((BR, 128), lambda i: (i, 0)),
            pl.BlockSpec((128, 256), lambda i: (0, 0)),
            pl.BlockSpec((128, 256), lambda i: (0, 0)),
            pl.BlockSpec((16, 256), lambda i: (0, 0)),
            pl.BlockSpec((128, 16), lambda i: (0, 0)),
            pl.BlockSpec((1, 256), lambda i: (0, 0)),
            pl.BlockSpec((256, 256), lambda i: (0, 0)),
            pl.BlockSpec((1, 256), lambda i: (0, 0)),
        ],
        out_specs=pl.BlockSpec((BR, 256), lambda i: (i, 0)),
        out_shape=jax.ShapeDtypeStruct((E_PAD // 16, 256), jnp.float32),
    )(esg, edg, w1i, w1j, wc, ssel, b1t, w2bd, b2t)


SCHUNK = 1024
SNCH = PER_W // SCHUNK
SNSTREAM = SCHUNK // SUB


def _sc_scatter_body(msg_hbm, dstT_hbm, zs_hbm, sums_hbm,
                     msgv, didx, acc, sem):
    cid = lax.axis_index("c")
    sid = lax.axis_index("s")
    wid = sid * NC + cid

    pltpu.sync_copy(zs_hbm, acc.at[pl.ds(sid * RPT, RPT)])
    plsc.subcore_barrier()

    def chunk(ci, carry):
        ebase = wid * PER_W + ci * SCHUNK
        pltpu.sync_copy(msg_hbm.at[pl.ds(ebase, SCHUNK)], msgv)
        pltpu.sync_copy(dstT_hbm.at[pl.ds(ebase, SCHUNK)], didx)
        pltpu.sync_copy(msgv, acc.at[didx], add=True)
        return carry

    lax.fori_loop(0, SNCH, chunk, 0)
    plsc.subcore_barrier()
    pltpu.sync_copy(acc.at[pl.ds(sid * RPT, RPT)],
                    sums_hbm.at[cid, pl.ds(sid * RPT, RPT)])


@functools.cache
def _sc_scatter():
    return pl.kernel(
        _sc_scatter_body,
        out_type=[jax.ShapeDtypeStruct((NC, NS, H), jnp.float32)],
        mesh=_sc_mesh(),
        compiler_params=_SC_PARAMS,
        scratch_types=[
            pltpu.VMEM((SCHUNK, H), jnp.float32),
            pltpu.VMEM((SCHUNK,), jnp.int32),
            pltpu.VMEM_SHARED((NS, H), jnp.float32),
            pltpu.SemaphoreType.DMA,
        ],
    )


CCHUNK = 1024
CNCH = PER_W // CCHUNK
CNSTREAM = CCHUNK // SUB


def _sc_count_body(dstT_hbm, ones_hbm, zc_hbm, cnts_hbm,
                   didx, onesv, accc, sem):
    cid = lax.axis_index("c")
    sid = lax.axis_index("s")
    wid = sid * NC + cid

    pltpu.sync_copy(zc_hbm, accc.at[pl.ds(sid * RPT, RPT)])
    pltpu.sync_copy(ones_hbm, onesv)
    plsc.subcore_barrier()

    def outer(ci, carry):
        ebase = wid * PER_W + ci * CCHUNK
        pltpu.sync_copy(dstT_hbm.at[pl.ds(ebase, CCHUNK)], didx)
        pltpu.sync_copy(onesv, accc.at[didx], add=True)
        return carry

    lax.fori_loop(0, CNCH, outer, 0)
    plsc.subcore_barrier()
    pltpu.sync_copy(accc.at[pl.ds(sid * RPT, RPT)],
                    cnts_hbm.at[cid, pl.ds(sid * RPT, RPT)])


@functools.cache
def _sc_count():
    return pl.kernel(
        _sc_count_body,
        out_type=[jax.ShapeDtypeStruct((NC, NS, H), jnp.float32)],
        mesh=_sc_mesh(),
        compiler_params=_SC_PARAMS,
        scratch_types=[
            pltpu.VMEM((CCHUNK,), jnp.int32),
            pltpu.VMEM((CCHUNK, H), jnp.float32),
            pltpu.VMEM_SHARED((NS, H), jnp.float32),
            pltpu.SemaphoreType.DMA,
        ],
    )


def _final_body(s0_ref, s1_ref, c0_ref, c1_ref, batch_ref, wp_ref, bp_ref,
                wz_ref, bz8_ref, s_ref, z_ref):
    tot = s0_ref[...] + s1_ref[...]
    cnt = jnp.maximum(c0_ref[...] + c1_ref[...], 1.0)
    h = jnp.maximum(tot / cnt, 0.0)
    logits = jnp.dot(h, wp_ref[...], preferred_element_type=jnp.float32) \
        + bp_ref[...]
    mx = jnp.max(logits, axis=1, keepdims=True)
    ex = jnp.exp(logits - mx)
    s = ex / jnp.sum(ex, axis=1, keepdims=True)
    s_ref[...] = s
    hz = jnp.dot(h, wz_ref[...], preferred_element_type=jnp.float32)
    wz8 = jnp.concatenate([s[:, 0:1] * hz, s[:, 1:2] * hz], axis=1)
    oh = (batch_ref[...] == lax.broadcasted_iota(jnp.int32, (BN, G), 1)
          ).astype(jnp.float32)
    part = lax.dot_general(oh, wz8, (((0,), (0,)), ((), ())),
                           preferred_element_type=jnp.float32)

    @pl.when(pl.program_id(0) == 0)
    def _():
        z_ref[...] = jnp.broadcast_to(bz8_ref[...], (G, 8))

    z_ref[...] += part


def _finalize(s0, s1, c0, c1, batchc, wp, bpr, wz, bz8):
    grid = (N // BN,)
    return pl.pallas_call(
        _final_body,
        grid=grid,
        in_specs=[
            pl.BlockSpec((BN, H), lambda i: (i, 0)),
            pl.BlockSpec((BN, H), lambda i: (i, 0)),
            pl.BlockSpec((BN, 1), lambda i: (i, 0)),
            pl.BlockSpec((BN, 1), lambda i: (i, 0)),
            pl.BlockSpec((BN, 1), lambda i: (i, 0)),
            pl.BlockSpec((H, 2), lambda i: (0, 0)),
            pl.BlockSpec((1, 2), lambda i: (0, 0)),
            pl.BlockSpec((H, 4), lambda i: (0, 0)),
            pl.BlockSpec((1, 8), lambda i: (0, 0)),
        ],
        out_specs=[
            pl.BlockSpec((BN, 2), lambda i: (i, 0)),
            pl.BlockSpec((G, 8), lambda i: (0, 0)),
        ],
        out_shape=[
            jax.ShapeDtypeStruct((N, 2), jnp.float32),
            jax.ShapeDtypeStruct((G, 8), jnp.float32),
        ],
    )(s0, s1, c0, c1, batchc, wp, bpr, wz, bz8)


def kernel(x, pos, edge_index, batch, W1, b1, W2, b2, Wp, bp, Wz, bz):
    pad = E_PAD - E
    src = edge_index[0]
    dst = edge_index[1]
    zpad = jnp.zeros((pad,), jnp.int32)
    srcp = jnp.concatenate([src, zpad])
    dstp = jnp.concatenate([dst, zpad])
    dstT = jnp.concatenate([dst, jnp.full((pad,), N, jnp.int32)])
    xp = jnp.concatenate([x, pos, jnp.zeros((N, 1), jnp.float32)], axis=1)

    es, ed = _sc_gather()(srcp, dstp, xp)
    esg = es.reshape(E_PAD // 16, 128)
    edg = ed.reshape(E_PAD // 16, 128)

    eye = jnp.eye(16, dtype=jnp.float32)
    zeros4 = jnp.zeros((4, H), jnp.float32)
    w1i = jnp.kron(eye, jnp.concatenate([W1[0:4], zeros4], axis=0))
    w1j = jnp.kron(eye, jnp.concatenate([W1[4:8], zeros4], axis=0))
    wc = jnp.kron(eye, W1[8:9])
    ssel = jnp.kron(eye, jnp.array([[0.], [0.], [0.], [0.],
                                    [1.], [1.], [1.], [0.]], jnp.float32))
    b1t = jnp.tile(b1, 16).reshape(1, 256)
    w2bd = jnp.kron(eye, W2)
    b2t = jnp.tile(b2, 16).reshape(1, 256)

    msgg = _edge_mlp(esg, edg, w1i, w1j, wc, ssel, b1t, w2bd, b2t)
    msg = msgg.reshape(E_PAD, H)

    ones = jnp.ones((CCHUNK, H), jnp.float32)
    zs = jnp.zeros((RPT, H), jnp.float32)
    zc = jnp.zeros((RPT, H), jnp.float32)
    (sums,) = _sc_scatter()(msg, dstT, zs)
    (cnts,) = _sc_count()(dstT, ones, zc)

    s, z8 = _finalize(sums[0, :N], sums[1, :N],
                      cnts[0, :N, 0:1], cnts[1, :N, 0:1],
                      batch.reshape(N, 1),
                      Wp, bp.reshape(1, 2), Wz,
                      jnp.concatenate([bz, bz]).reshape(1, 8))
    z = z8.reshape(G, 2, 4)
    return (z, s)
